# baseline (speedup 1.0000x reference)
import functools

import jax
import jax.numpy as jnp
from jax import lax
from jax.experimental import pallas as pl
from jax.experimental.pallas import tpu as pltpu
from jax.experimental.pallas import tpu_sc as plsc

BATCH = 4096
HIST = 50
EMBED_DIM = 32

_INFO = plsc.get_sparse_core_info()
NC = _INFO.num_cores
NS = _INFO.num_subcores
NW = NC * NS
CHUNK = BATCH // NW
N_CHUNKS = HIST
NBUF = 5
ROUNDS = N_CHUNKS // NBUF
DRAIN_SLACK = 2
LANES = 16

_MESH = plsc.VectorSubcoreMesh(core_axis_name="c", subcore_axis_name="s")


@functools.partial(
    pl.kernel,
    out_type=jax.ShapeDtypeStruct((HIST, EMBED_DIM, BATCH), jnp.float32),
    mesh=_MESH,
    scratch_types=[
        pltpu.VMEM((N_CHUNKS, CHUNK), jnp.int32),
        pltpu.VMEM((NBUF, CHUNK, EMBED_DIM), jnp.float32),
        pltpu.VMEM((NBUF, EMBED_DIM, CHUNK), jnp.float32),
        pltpu.SemaphoreType.DMA((NBUF,)),
        pltpu.SemaphoreType.DMA((NBUF,)),
    ],
    compiler_params=pltpu.CompilerParams(
        use_tc_tiling_on_sc=False, needs_layout_passes=False
    ),
)
def _sc_gather(idx_hbm, table_hbm, out_hbm, idx_v, rows_v, trans_v, sem_g, sem_o):
    wid = lax.axis_index("s") * NC + lax.axis_index("c")
    b0 = wid * CHUNK
    pltpu.sync_copy(idx_hbm.at[:, pl.ds(b0, CHUNK)], idx_v)

    def gather(j, b):
        return pltpu.async_copy(
            table_hbm.at[idx_v.at[j]], rows_v.at[b], sem_g.at[b]
        )

    def copy_out(j, b):
        return pltpu.make_async_copy(
            trans_v.at[b],
            out_hbm.at[j, :, pl.ds(b0, CHUNK)],
            sem_o.at[b],
        )

    def transpose(b):
        rows = rows_v.at[b]
        trans = trans_v.at[b]
        lane = lax.iota(jnp.int32, LANES)
        for g in range(CHUNK // LANES):
            base = lane + g * LANES
            for e in range(EMBED_DIM):
                col = jnp.full((LANES,), e, jnp.int32)
                v = plsc.load_gather(rows, [base, col])
                trans[e, pl.ds(g * LANES, LANES)] = v

    for b in range(NBUF):
        gather(b, b)

    @pl.loop(0, ROUNDS)
    def _(r):
        for b in range(NBUF):
            j = r * NBUF + b
            pltpu.make_async_copy(
                table_hbm.at[idx_v.at[j]], rows_v.at[b], sem_g.at[b]
            ).wait()
            transpose(b)
            copy_out(j, b).start()
            bn = (b - DRAIN_SLACK) % NBUF
            jo = r * NBUF + b - DRAIN_SLACK
            jn = jo + NBUF

            @pl.when((jo >= 0) & (jn < N_CHUNKS))
            def _():
                copy_out(jo, bn).wait()
                gather(jn, bn)

    for b in range(NBUF):
        j = N_CHUNKS - NBUF + b
        copy_out(j, b).wait()


def kernel(indices, embedding):
    idx_t = jnp.transpose(indices.astype(jnp.int32))
    out = _sc_gather(idx_t, embedding)
    return jnp.transpose(out, (2, 0, 1))


# speedup vs baseline: 1.1833x; 1.1833x over previous
import functools

import jax
import jax.numpy as jnp
from jax import lax
from jax.experimental import pallas as pl
from jax.experimental.pallas import tpu as pltpu
from jax.experimental.pallas import tpu_sc as plsc

BATCH = 4096
HIST = 50
EMBED_DIM = 32

_INFO = plsc.get_sparse_core_info()
NC = _INFO.num_cores
NS = _INFO.num_subcores
NW = NC * NS
CHUNK = BATCH // NW
N_CHUNKS = HIST
NBUF = 5
ROUNDS = N_CHUNKS // NBUF
DRAIN_SLACK = 2
LANES = 16
CHUNK_PAD = CHUNK + 5

_MESH = plsc.VectorSubcoreMesh(core_axis_name="c", subcore_axis_name="s")


@functools.partial(
    pl.kernel,
    out_type=jax.ShapeDtypeStruct((HIST, EMBED_DIM, BATCH), jnp.float32),
    mesh=_MESH,
    scratch_types=[
        pltpu.VMEM((N_CHUNKS, CHUNK), jnp.int32),
        pltpu.VMEM((NBUF, CHUNK, EMBED_DIM), jnp.float32),
        pltpu.VMEM((NBUF, EMBED_DIM, CHUNK_PAD), jnp.float32),
        pltpu.SemaphoreType.DMA((NBUF,)),
        pltpu.SemaphoreType.DMA((NBUF,)),
    ],
    compiler_params=pltpu.CompilerParams(
        use_tc_tiling_on_sc=False, needs_layout_passes=False
    ),
)
def _sc_gather(idx_hbm, table_hbm, out_hbm, idx_v, rows_v, trans_v, sem_g, sem_o):
    wid = lax.axis_index("s") * NC + lax.axis_index("c")
    b0 = wid * CHUNK
    pltpu.sync_copy(idx_hbm.at[:, pl.ds(b0, CHUNK)], idx_v)

    def gather(j, b):
        return pltpu.async_copy(
            table_hbm.at[idx_v.at[j]], rows_v.at[b], sem_g.at[b]
        )

    def copy_out(j, b):
        return pltpu.make_async_copy(
            trans_v.at[b, :, pl.ds(0, CHUNK)],
            out_hbm.at[j, :, pl.ds(b0, CHUNK)],
            sem_o.at[b],
        )

    def transpose(b):
        rows = rows_v.at[b]
        trans = trans_v.at[b]
        e_lo = lax.iota(jnp.int32, LANES)
        e_hi = e_lo + LANES
        for c in range(CHUNK):
            col = jnp.full((LANES,), c, jnp.int32)
            v0 = rows[c, pl.ds(0, LANES)]
            v1 = rows[c, pl.ds(LANES, LANES)]
            plsc.store_scatter(trans, [e_lo, col], v0)
            plsc.store_scatter(trans, [e_hi, col], v1)

    for b in range(NBUF):
        gather(b, b)

    @pl.loop(0, ROUNDS)
    def _(r):
        for b in range(NBUF):
            j = r * NBUF + b
            pltpu.make_async_copy(
                table_hbm.at[idx_v.at[j]], rows_v.at[b], sem_g.at[b]
            ).wait()
            transpose(b)
            copy_out(j, b).start()
            bn = (b - DRAIN_SLACK) % NBUF
            jo = r * NBUF + b - DRAIN_SLACK
            jn = jo + NBUF

            @pl.when((jo >= 0) & (jn < N_CHUNKS))
            def _():
                copy_out(jo, bn).wait()
                gather(jn, bn)

    for b in range(NBUF):
        j = N_CHUNKS - NBUF + b
        copy_out(j, b).wait()


def kernel(indices, embedding):
    idx_t = jnp.transpose(indices.astype(jnp.int32))
    out = _sc_gather(idx_t, embedding)
    return jnp.transpose(out, (2, 0, 1))


# speedup vs baseline: 1.2411x; 1.0488x over previous
import functools

import jax
import jax.numpy as jnp
from jax import lax
from jax.experimental import pallas as pl
from jax.experimental.pallas import tpu as pltpu
from jax.experimental.pallas import tpu_sc as plsc

BATCH = 4096
HIST = 50
EMBED_DIM = 32

_INFO = plsc.get_sparse_core_info()
NC = _INFO.num_cores
NS = _INFO.num_subcores
NW = NC * NS
CHUNK = BATCH // NW
N_CHUNKS = HIST
NBUF = 5
ROUNDS = N_CHUNKS // NBUF
DRAIN_SLACK = 2
LANES = 16
CHUNK_PAD = CHUNK + 5

_MESH = plsc.VectorSubcoreMesh(core_axis_name="c", subcore_axis_name="s")


@functools.partial(
    pl.kernel,
    out_type=jax.ShapeDtypeStruct(
        (HIST, EMBED_DIM // 8, BATCH // CHUNK, 8, CHUNK), jnp.float32
    ),
    mesh=_MESH,
    scratch_types=[
        pltpu.VMEM((N_CHUNKS, CHUNK), jnp.int32),
        pltpu.VMEM((NBUF, CHUNK, EMBED_DIM), jnp.float32),
        pltpu.VMEM((NBUF, EMBED_DIM // 8, 8, CHUNK_PAD), jnp.float32),
        pltpu.SemaphoreType.DMA((NBUF,)),
        pltpu.SemaphoreType.DMA((NBUF,)),
    ],
    compiler_params=pltpu.CompilerParams(
        use_tc_tiling_on_sc=False, needs_layout_passes=False
    ),
)
def _sc_gather(idx_hbm, table_hbm, out_hbm, idx_v, rows_v, trans_v, sem_g, sem_o):
    wid = lax.axis_index("s") * NC + lax.axis_index("c")
    b0 = wid * CHUNK
    pltpu.sync_copy(idx_hbm.at[:, pl.ds(b0, CHUNK)], idx_v)

    def gather(j, b):
        return pltpu.async_copy(
            table_hbm.at[idx_v.at[j]], rows_v.at[b], sem_g.at[b]
        )

    def copy_out(j, b):
        return pltpu.make_async_copy(
            trans_v.at[b, :, :, pl.ds(0, CHUNK)],
            out_hbm.at[j, :, wid, :, :],
            sem_o.at[b],
        )

    def transpose(b):
        rows = rows_v.at[b]
        trans = trans_v.at[b]
        e_lo = lax.iota(jnp.int32, LANES)
        e_hi = e_lo + LANES
        r_lo, s_lo = e_lo >> 3, e_lo & 7
        r_hi, s_hi = e_hi >> 3, e_hi & 7
        for c in range(CHUNK):
            col = jnp.full((LANES,), c, jnp.int32)
            v0 = rows[c, pl.ds(0, LANES)]
            v1 = rows[c, pl.ds(LANES, LANES)]
            plsc.store_scatter(trans, [r_lo, s_lo, col], v0)
            plsc.store_scatter(trans, [r_hi, s_hi, col], v1)

    for b in range(NBUF):
        gather(b, b)

    @pl.loop(0, ROUNDS)
    def _(r):
        for b in range(NBUF):
            j = r * NBUF + b
            pltpu.make_async_copy(
                table_hbm.at[idx_v.at[j]], rows_v.at[b], sem_g.at[b]
            ).wait()
            transpose(b)
            copy_out(j, b).start()
            bn = (b - DRAIN_SLACK) % NBUF
            jo = r * NBUF + b - DRAIN_SLACK
            jn = jo + NBUF

            @pl.when((jo >= 0) & (jn < N_CHUNKS))
            def _():
                copy_out(jo, bn).wait()
                gather(jn, bn)

    for b in range(NBUF):
        j = N_CHUNKS - NBUF + b
        copy_out(j, b).wait()


def kernel(indices, embedding):
    idx_t = jnp.transpose(indices.astype(jnp.int32))
    out5 = _sc_gather(idx_t, embedding)
    out = jnp.transpose(out5, (2, 4, 0, 1, 3))
    return out.reshape(BATCH, HIST, EMBED_DIM)
